# baseline (speedup 1.0000x reference)
import functools

import jax
import jax.numpy as jnp
from jax import lax
from jax.experimental import pallas as pl
from jax.experimental.pallas import tpu as pltpu
from jax.experimental.pallas import tpu_sc as plsc

_B, _P, _G, _C = 32, 16800, 16, 2
_TH = 0.35
_NEGPOS = 7
_V0, _V1 = 0.1, 0.2


def _smooth_l1(d):
    ad = jnp.abs(d)
    return jnp.where(ad < 1.0, 0.5 * d * d, ad - 0.5)


def _phase1_body(cls_ref, loc_ref, landm_ref, pri_ref, tgt_ref,
                 v_ref, sums_ref, s_ref):
    G, P = _G, _P

    @pl.when(pl.program_id(0) == 0)
    def _init():
        pri = pri_ref[...]
        pcx, pcy = pri[0:1], pri[1:2]
        pw, ph = pri[2:3], pri[3:4]
        s_ref[0:1] = pcx - pw * 0.5
        s_ref[1:2] = pcy - ph * 0.5
        s_ref[2:3] = pcx + pw * 0.5
        s_ref[3:4] = pcy + ph * 0.5
        s_ref[4:5] = pw * ph
        inv_vw = 1.0 / (_V0 * pw)
        inv_vh = 1.0 / (_V0 * ph)
        s_ref[5:6] = inv_vw
        s_ref[6:7] = inv_vh
        s_ref[7:8] = 1.0 / pw
        s_ref[8:9] = 1.0 / ph
        s_ref[9:19] = jnp.concatenate([pcx, pcy] * 5, axis=0)
        s_ref[19:29] = jnp.concatenate([inv_vw, inv_vh] * 5, axis=0)

    pri = pri_ref[...]
    pcx, pcy = pri[0:1], pri[1:2]
    pw, ph = pri[2:3], pri[3:4]
    px1, py1 = s_ref[0:1], s_ref[1:2]
    px2, py2 = s_ref[2:3], s_ref[3:4]

    tgt = tgt_ref[0]
    tx1, ty1 = tgt[:, 0:1], tgt[:, 1:2]
    tx2, ty2 = tgt[:, 2:3], tgt[:, 3:4]

    iw = jnp.maximum(jnp.minimum(tx2, px2) - jnp.maximum(tx1, px1), 0.0)
    ih = jnp.maximum(jnp.minimum(ty2, py2) - jnp.maximum(ty1, py1), 0.0)
    inter = iw * ih
    area_t = (tx2 - tx1) * (ty2 - ty1)
    ov = inter / (area_t + s_ref[4:5] - inter)

    gi = lax.broadcasted_iota(jnp.int32, (G, P), 0)
    pi = lax.broadcasted_iota(jnp.int32, (G, P), 1)

    bto = jnp.max(ov, axis=0, keepdims=True)
    bti = jnp.min(jnp.where(ov == bto, gi, G), axis=0, keepdims=True)
    bpo = jnp.max(ov, axis=1, keepdims=True)
    bpi = jnp.min(jnp.where(ov == bpo, pi, P), axis=1, keepdims=True)
    valid = bpo >= 0.2

    eqbp = bpi == pi
    genc = jnp.max(jnp.where(eqbp, 2 * gi + valid.astype(jnp.int32), -1),
                   axis=0, keepdims=True)
    has = genc >= 0
    gidx = lax.shift_right_logical(jnp.maximum(genc, 0), 1)
    bto2 = jnp.where(has & (jnp.bitwise_and(genc, 1) == 1), 2.0, bto)
    bti2 = jnp.where(has, gidx, bti)

    onehot = (gi == bti2).astype(jnp.float32)
    m = lax.dot_general(tgt, onehot, (((0,), (0,)), ((), ())),
                        precision=lax.Precision.HIGHEST,
                        preferred_element_type=jnp.float32)

    label = m[14:15]
    conf = jnp.where(bto2 < _TH, 0, label.astype(jnp.int32))
    posf = (conf != 0).astype(jnp.float32)

    mx1, my1, mx2, my2 = m[0:1], m[1:2], m[2:3], m[3:4]
    gcx = ((mx1 + mx2) * 0.5 - pcx) * s_ref[5:6]
    gcy = ((my1 + my2) * 0.5 - pcy) * s_ref[6:7]
    gw = jnp.log((mx2 - mx1) * s_ref[7:8]) * (1.0 / _V1)
    gh = jnp.log((my2 - my1) * s_ref[8:9]) * (1.0 / _V1)
    gloc = jnp.concatenate([gcx, gcy, gw, gh], axis=0)
    loss_l = jnp.sum(_smooth_l1(loc_ref[0] - gloc) * posf)

    glm = (m[4:14] - s_ref[9:19]) * s_ref[19:29]
    loss_lm = jnp.sum(_smooth_l1(landm_ref[0] - glm) * posf)

    cls0, cls1 = cls_ref[0, 0:1], cls_ref[0, 1:2]
    mc = jnp.maximum(cls0, cls1)
    lse = mc + jnp.log(jnp.exp(cls0 - mc) + jnp.exp(cls1 - mc))
    csel = jnp.where(conf == 0, cls0, cls1)
    v = lse - csel

    num_pos = jnp.sum(posf)

    v_ref[0] = jnp.where(conf != 0, -v, v)
    li = lax.broadcasted_iota(jnp.int32, (1, 128), 1)
    sums_ref[0] = jnp.where(
        li == 0, loss_l, jnp.where(li == 1, loss_lm,
                                   jnp.where(li == 2, num_pos, 0.0)))


_L = 16
_CH = _P // _L
_NC = 2


def _mine_body(v_hbm, sums_hbm, out_hbm, vbuf, vbbuf, sbuf, hist, obuf,
               nrows=_B):
    wid = lax.axis_index("s") * _NC + lax.axis_index("c")
    wid = lax.rem(wid, nrows)
    pltpu.sync_copy(v_hbm.at[wid], vbuf)
    pltpu.sync_copy(sums_hbm.at[wid], sbuf)

    k = jnp.minimum(sbuf[pl.ds(0, _L)][2].astype(jnp.int32) * _NEGPOS, _P - 1)

    zeros16 = jnp.zeros((_L,), jnp.int32)
    ones16 = jnp.ones((_L,), jnp.int32)
    lane = lax.iota(jnp.int32, _L)

    def zero_hist(nbins):
        def zb(j, c):
            hist[pl.ds(j * _L, _L)] = zeros16
            return c
        lax.fori_loop(0, nbins // _L, zb, 0)

    def scan_level(nbins, kr):
        nch = nbins // _L

        def st1(jj, c):
            acc, found, j_star, acc_b = c
            j = nch - 1 - jj
            ssum = jnp.sum(hist[pl.ds(j * _L, _L)])
            newacc = acc + ssum
            hit = jnp.logical_and(found == 0, newacc >= kr)
            j_star = jnp.where(hit, j, j_star)
            acc_b = jnp.where(hit, acc, acc_b)
            found = jnp.where(newacc >= kr, 1, found)
            return (newacc, found, j_star, acc_b)

        _, _, j_star, acc_b = lax.fori_loop(0, nch, st1, (0, 0, 0, 0))
        chr_ = lax.rev(hist[pl.ds(j_star * _L, _L)], (0,))
        cum = plsc.cumsum(chr_)
        mask = (acc_b + cum) >= kr
        l_star = jnp.max(plsc.all_reduce_ffs(mask))
        f_star = j_star * _L + (_L - 1) - l_star
        hsel = jnp.sum(jnp.where(lane == l_star, chr_, 0))
        csel = jnp.sum(jnp.where(lane == l_star, cum, 0))
        return f_star, acc_b + csel - hsel

    zero_hist(2048)

    def pa(i, c):
        for u in range(2):
            x = vbuf[pl.ds((2 * i + u) * _L, _L)]
            vb = jnp.bitwise_and(lax.bitcast_convert_type(x, jnp.int32),
                                 0x7FFFFFFF)
            vbbuf[pl.ds((2 * i + u) * _L, _L)] = vb
            plsc.addupdate_scatter(
                hist, [lax.shift_right_logical(vb, 20)], ones16)
        return c
    lax.fori_loop(0, _CH // 2, pa, 0)
    f1, above1 = scan_level(2048, k)

    zero_hist(2048)

    def pb(i, c):
        for u in range(2):
            vb = vbbuf[pl.ds((2 * i + u) * _L, _L)]
            m = lax.shift_right_logical(vb, 20) == f1
            b2 = jnp.bitwise_and(lax.shift_right_logical(vb, 9), 0x7FF)
            plsc.addupdate_scatter(hist, [b2], ones16, mask=m)
        return c
    lax.fori_loop(0, _CH // 2, pb, 0)
    f2, above2 = scan_level(2048, k - above1)
    prefix = jnp.bitwise_or(lax.shift_left(f1, 11), f2)

    zero_hist(512)

    def pc(i, c):
        for u in range(2):
            vb = vbbuf[pl.ds((2 * i + u) * _L, _L)]
            m = lax.shift_right_logical(vb, 9) == prefix
            b3 = jnp.bitwise_and(vb, 0x1FF)
            plsc.addupdate_scatter(hist, [b3], ones16, mask=m)
        return c
    lax.fori_loop(0, _CH // 2, pc, 0)
    f3, cnt_gt = scan_level(512, k - above1 - above2)
    cnt_gt = cnt_gt + above1 + above2
    t = jnp.bitwise_or(lax.shift_left(prefix, 9), f3)
    r = k - cnt_gt

    def pd(i, c):
        accv, tcnt, tnp = c
        for u in range(2):
            sl = pl.ds((2 * i + u) * _L, _L)
            raw = lax.bitcast_convert_type(vbuf[sl], jnp.int32)
            pos = raw < 0
            vb = vbbuf[sl]
            gt = vb > t
            tie = vb == t
            x = lax.bitcast_convert_type(vb, jnp.float32)
            accv = accv + jnp.where(jnp.logical_or(gt, pos), x, 0.0)
            tcnt = tcnt + jnp.where(tie, 1, 0)
            tnp = tnp + jnp.where(jnp.logical_and(tie, jnp.logical_not(pos)),
                                  1, 0)
        return accv, tcnt, tnp

    accv, tcnt, tnp = lax.fori_loop(
        0, _CH // 2, pd, (jnp.zeros((_L,), jnp.float32), zeros16, zeros16))
    m_ties = jnp.sum(tcnt)
    all_tnp = jnp.sum(tnp)

    def pe_loop():
        def pe(i, c):
            base, cnt = c
            vb = vbbuf[pl.ds(i * _L, _L)]
            pos = lax.bitcast_convert_type(vbuf[pl.ds(i * _L, _L)],
                                           jnp.int32) < 0
            ti = jnp.where(vb == t, 1, 0)
            excl = plsc.cumsum(ti) - ti
            seltie = jnp.logical_and(
                jnp.logical_and(ti == 1, base + excl < r),
                jnp.logical_not(pos))
            cnt = cnt + jnp.sum(jnp.where(seltie, 1, 0))
            base = base + jnp.sum(ti)
            return base, cnt
        _, cnt = lax.fori_loop(0, _CH, pe, (0, 0))
        return cnt

    n_tie_nonpos = lax.cond(r == m_ties, lambda: all_tnp, pe_loop)

    tf = jnp.full((_L,), t, jnp.int32)
    tval = lax.bitcast_convert_type(tf, jnp.float32) * n_tie_nonpos.astype(jnp.float32)
    total = jnp.sum(accv) + jnp.max(tval)
    obuf[...] = jnp.full((_L,), total, jnp.float32)
    pltpu.sync_copy(obuf, out_hbm.at[wid])


def _phase1_call(cls_t, loc_t, landm_t, pri_t, tgt, nb):
    P = _P
    return pl.pallas_call(
        _phase1_body,
        grid=(nb,),
        in_specs=[
            pl.BlockSpec((1, _C, P), lambda b: (b, 0, 0)),
            pl.BlockSpec((1, 4, P), lambda b: (b, 0, 0)),
            pl.BlockSpec((1, 10, P), lambda b: (b, 0, 0)),
            pl.BlockSpec((4, P), lambda b: (0, 0)),
            pl.BlockSpec((1, _G, 15), lambda b: (b, 0, 0)),
        ],
        out_specs=[
            pl.BlockSpec((1, 1, P), lambda b: (b, 0, 0)),
            pl.BlockSpec((1, 1, 128), lambda b: (b, 0, 0)),
        ],
        out_shape=[
            jax.ShapeDtypeStruct((nb, 1, P), jnp.float32),
            jax.ShapeDtypeStruct((nb, 1, 128), jnp.float32),
        ],
        scratch_shapes=[pltpu.VMEM((32, P), jnp.float32)],
    )(cls_t, loc_t, landm_t, pri_t, tgt)


def _mine_call(vrows, smrows, nrows):
    mine = pl.kernel(
        functools.partial(_mine_body, nrows=nrows),
        mesh=plsc.VectorSubcoreMesh(core_axis_name="c", subcore_axis_name="s"),
        out_type=jax.ShapeDtypeStruct((nrows, _L), jnp.float32),
        compiler_params=pltpu.CompilerParams(needs_layout_passes=False),
        scratch_types=[
            pltpu.VMEM((_P,), jnp.float32),
            pltpu.VMEM((_P,), jnp.int32),
            pltpu.VMEM((128,), jnp.float32),
            pltpu.VMEM((2048,), jnp.int32),
            pltpu.VMEM((_L,), jnp.float32),
        ],
    )
    return mine(vrows, smrows)


@jax.jit
def kernel(cls_data, loc_data, landm_data, priors, targets):
    B, P = _B, _P
    H = B // 2
    cls_t = jnp.transpose(cls_data, (0, 2, 1))
    loc_t = jnp.transpose(loc_data, (0, 2, 1))
    landm_t = jnp.transpose(landm_data, (0, 2, 1))
    pri_t = jnp.transpose(priors, (1, 0))

    va, sa = _phase1_call(cls_t[:H], loc_t[:H], landm_t[:H], pri_t,
                          targets[:H], H)
    ra = _mine_call(va.reshape(H, P), sa.reshape(H, 128), H)
    vb, sb = _phase1_call(cls_t[H:], loc_t[H:], landm_t[H:], pri_t,
                          targets[H:], H)
    rb = _mine_call(vb.reshape(H, P), sb.reshape(H, 128), H)

    sm = jnp.concatenate([sa.reshape(H, 128), sb.reshape(H, 128)], axis=0)
    loss_c = jnp.sum(ra[:, 0]) + jnp.sum(rb[:, 0])
    n = jnp.maximum(jnp.sum(sm[:, 2]), 1.0)
    return (jnp.sum(sm[:, 0]) / n, loss_c / n, jnp.sum(sm[:, 1]) / n)


# speedup vs baseline: 1.1233x; 1.1233x over previous
import functools

import jax
import jax.numpy as jnp
from jax import lax
from jax.experimental import pallas as pl
from jax.experimental.pallas import tpu as pltpu
from jax.experimental.pallas import tpu_sc as plsc

_B, _P, _G, _C = 32, 16800, 16, 2
_TH = 0.35
_NEGPOS = 7
_V0, _V1 = 0.1, 0.2


def _smooth_l1(d):
    ad = jnp.abs(d)
    return jnp.where(ad < 1.0, 0.5 * d * d, ad - 0.5)


def _phase1_body(cls_ref, loc_ref, landm_ref, pri_ref, tgt_ref,
                 v_ref, sums_ref, s_ref):
    G, P = _G, _P

    @pl.when(pl.program_id(0) == 0)
    def _init():
        pri = pri_ref[...]
        pcx, pcy = pri[0:1], pri[1:2]
        pw, ph = pri[2:3], pri[3:4]
        s_ref[0:1] = pcx - pw * 0.5
        s_ref[1:2] = pcy - ph * 0.5
        s_ref[2:3] = pcx + pw * 0.5
        s_ref[3:4] = pcy + ph * 0.5
        s_ref[4:5] = pw * ph
        inv_vw = 1.0 / (_V0 * pw)
        inv_vh = 1.0 / (_V0 * ph)
        s_ref[5:6] = inv_vw
        s_ref[6:7] = inv_vh
        s_ref[7:8] = 1.0 / pw
        s_ref[8:9] = 1.0 / ph
        s_ref[9:19] = jnp.concatenate([pcx, pcy] * 5, axis=0)
        s_ref[19:29] = jnp.concatenate([inv_vw, inv_vh] * 5, axis=0)

    pri = pri_ref[...]
    pcx, pcy = pri[0:1], pri[1:2]
    pw, ph = pri[2:3], pri[3:4]
    px1, py1 = s_ref[0:1], s_ref[1:2]
    px2, py2 = s_ref[2:3], s_ref[3:4]

    tgt = tgt_ref[0]
    tx1, ty1 = tgt[:, 0:1], tgt[:, 1:2]
    tx2, ty2 = tgt[:, 2:3], tgt[:, 3:4]

    iw = jnp.maximum(jnp.minimum(tx2, px2) - jnp.maximum(tx1, px1), 0.0)
    ih = jnp.maximum(jnp.minimum(ty2, py2) - jnp.maximum(ty1, py1), 0.0)
    inter = iw * ih
    area_t = (tx2 - tx1) * (ty2 - ty1)
    ov = inter / (area_t + s_ref[4:5] - inter)

    gi = lax.broadcasted_iota(jnp.int32, (G, P), 0)
    pi = lax.broadcasted_iota(jnp.int32, (G, P), 1)

    bto = jnp.max(ov, axis=0, keepdims=True)
    bti = jnp.min(jnp.where(ov == bto, gi, G), axis=0, keepdims=True)
    bpo = jnp.max(ov, axis=1, keepdims=True)
    bpi = jnp.min(jnp.where(ov == bpo, pi, P), axis=1, keepdims=True)
    valid = bpo >= 0.2

    eqbp = bpi == pi
    genc = jnp.max(jnp.where(eqbp, 2 * gi + valid.astype(jnp.int32), -1),
                   axis=0, keepdims=True)
    has = genc >= 0
    gidx = lax.shift_right_logical(jnp.maximum(genc, 0), 1)
    bto2 = jnp.where(has & (jnp.bitwise_and(genc, 1) == 1), 2.0, bto)
    bti2 = jnp.where(has, gidx, bti)

    onehot = (gi == bti2).astype(jnp.float32)
    m = lax.dot_general(tgt, onehot, (((0,), (0,)), ((), ())),
                        precision=lax.Precision.HIGHEST,
                        preferred_element_type=jnp.float32)

    label = m[14:15]
    conf = jnp.where(bto2 < _TH, 0, label.astype(jnp.int32))
    posf = (conf != 0).astype(jnp.float32)

    mx1, my1, mx2, my2 = m[0:1], m[1:2], m[2:3], m[3:4]
    gcx = ((mx1 + mx2) * 0.5 - pcx) * s_ref[5:6]
    gcy = ((my1 + my2) * 0.5 - pcy) * s_ref[6:7]
    gw = jnp.log((mx2 - mx1) * s_ref[7:8]) * (1.0 / _V1)
    gh = jnp.log((my2 - my1) * s_ref[8:9]) * (1.0 / _V1)
    gloc = jnp.concatenate([gcx, gcy, gw, gh], axis=0)
    loss_l = jnp.sum(_smooth_l1(loc_ref[0] - gloc) * posf)

    glm = (m[4:14] - s_ref[9:19]) * s_ref[19:29]
    loss_lm = jnp.sum(_smooth_l1(landm_ref[0] - glm) * posf)

    cls0, cls1 = cls_ref[0, 0:1], cls_ref[0, 1:2]
    mc = jnp.maximum(cls0, cls1)
    lse = mc + jnp.log(jnp.exp(cls0 - mc) + jnp.exp(cls1 - mc))
    csel = jnp.where(conf == 0, cls0, cls1)
    v = lse - csel

    num_pos = jnp.sum(posf)

    v_ref[0] = jnp.where(conf != 0, -v, v)
    li = lax.broadcasted_iota(jnp.int32, (1, 128), 1)
    sums_ref[0] = jnp.where(
        li == 0, loss_l, jnp.where(li == 1, loss_lm,
                                   jnp.where(li == 2, num_pos, 0.0)))


_L = 16
_CH = _P // _L
_NC = 2


def _mine_body(v_hbm, sums_hbm, out_hbm, vbuf, vbbuf, sbuf, hist, obuf,
               nrows=_B):
    wid = lax.axis_index("s") * _NC + lax.axis_index("c")
    wid = lax.rem(wid, nrows)
    pltpu.sync_copy(v_hbm.at[wid], vbuf)
    pltpu.sync_copy(sums_hbm.at[wid], sbuf)

    k = jnp.minimum(sbuf[pl.ds(0, _L)][2].astype(jnp.int32) * _NEGPOS, _P - 1)

    zeros16 = jnp.zeros((_L,), jnp.int32)
    ones16 = jnp.ones((_L,), jnp.int32)
    lane = lax.iota(jnp.int32, _L)

    def zero_hist(nbins):
        def zb(j, c):
            hist[pl.ds(j * _L, _L)] = zeros16
            return c
        lax.fori_loop(0, nbins // _L, zb, 0)

    def scan_level(nbins, kr):
        nch = nbins // _L

        def st1(jj, c):
            acc, found, j_star, acc_b = c
            j = nch - 1 - jj
            ssum = jnp.sum(hist[pl.ds(j * _L, _L)])
            newacc = acc + ssum
            hit = jnp.logical_and(found == 0, newacc >= kr)
            j_star = jnp.where(hit, j, j_star)
            acc_b = jnp.where(hit, acc, acc_b)
            found = jnp.where(newacc >= kr, 1, found)
            return (newacc, found, j_star, acc_b)

        _, _, j_star, acc_b = lax.fori_loop(0, nch, st1, (0, 0, 0, 0))
        chr_ = lax.rev(hist[pl.ds(j_star * _L, _L)], (0,))
        cum = plsc.cumsum(chr_)
        mask = (acc_b + cum) >= kr
        l_star = jnp.max(plsc.all_reduce_ffs(mask))
        f_star = j_star * _L + (_L - 1) - l_star
        hsel = jnp.sum(jnp.where(lane == l_star, chr_, 0))
        csel = jnp.sum(jnp.where(lane == l_star, cum, 0))
        return f_star, acc_b + csel - hsel

    zero_hist(2048)

    def pa(i, c):
        for u in range(6):
            x = vbuf[pl.ds((6 * i + u) * _L, _L)]
            vb = jnp.bitwise_and(lax.bitcast_convert_type(x, jnp.int32),
                                 0x7FFFFFFF)
            vbbuf[pl.ds((6 * i + u) * _L, _L)] = vb
            plsc.addupdate_scatter(
                hist, [lax.shift_right_logical(vb, 20)], ones16)
        return c
    lax.fori_loop(0, _CH // 6, pa, 0)
    f1, above1 = scan_level(2048, k)

    zero_hist(2048)

    def pb(i, c):
        for u in range(6):
            vb = vbbuf[pl.ds((6 * i + u) * _L, _L)]
            m = lax.shift_right_logical(vb, 20) == f1
            b2 = jnp.bitwise_and(lax.shift_right_logical(vb, 9), 0x7FF)
            plsc.addupdate_scatter(hist, [b2], ones16, mask=m)
        return c
    lax.fori_loop(0, _CH // 6, pb, 0)
    f2, above2 = scan_level(2048, k - above1)
    prefix = jnp.bitwise_or(lax.shift_left(f1, 11), f2)

    zero_hist(512)

    def pc(i, c):
        for u in range(6):
            vb = vbbuf[pl.ds((6 * i + u) * _L, _L)]
            m = lax.shift_right_logical(vb, 9) == prefix
            b3 = jnp.bitwise_and(vb, 0x1FF)
            plsc.addupdate_scatter(hist, [b3], ones16, mask=m)
        return c
    lax.fori_loop(0, _CH // 6, pc, 0)
    f3, cnt_gt = scan_level(512, k - above1 - above2)
    cnt_gt = cnt_gt + above1 + above2
    t = jnp.bitwise_or(lax.shift_left(prefix, 9), f3)
    r = k - cnt_gt

    def pd(i, c):
        accv, tcnt, tnp = c
        for u in range(6):
            sl = pl.ds((6 * i + u) * _L, _L)
            raw = lax.bitcast_convert_type(vbuf[sl], jnp.int32)
            pos = raw < 0
            vb = vbbuf[sl]
            gt = vb > t
            tie = vb == t
            x = lax.bitcast_convert_type(vb, jnp.float32)
            accv = accv + jnp.where(jnp.logical_or(gt, pos), x, 0.0)
            tcnt = tcnt + jnp.where(tie, 1, 0)
            tnp = tnp + jnp.where(jnp.logical_and(tie, jnp.logical_not(pos)),
                                  1, 0)
        return accv, tcnt, tnp

    accv, tcnt, tnp = lax.fori_loop(
        0, _CH // 6, pd, (jnp.zeros((_L,), jnp.float32), zeros16, zeros16))
    m_ties = jnp.sum(tcnt)
    all_tnp = jnp.sum(tnp)

    def pe_loop():
        def pe(i, c):
            base, cnt = c
            vb = vbbuf[pl.ds(i * _L, _L)]
            pos = lax.bitcast_convert_type(vbuf[pl.ds(i * _L, _L)],
                                           jnp.int32) < 0
            ti = jnp.where(vb == t, 1, 0)
            excl = plsc.cumsum(ti) - ti
            seltie = jnp.logical_and(
                jnp.logical_and(ti == 1, base + excl < r),
                jnp.logical_not(pos))
            cnt = cnt + jnp.sum(jnp.where(seltie, 1, 0))
            base = base + jnp.sum(ti)
            return base, cnt
        _, cnt = lax.fori_loop(0, _CH, pe, (0, 0))
        return cnt

    n_tie_nonpos = lax.cond(r == m_ties, lambda: all_tnp, pe_loop)

    tf = jnp.full((_L,), t, jnp.int32)
    tval = lax.bitcast_convert_type(tf, jnp.float32) * n_tie_nonpos.astype(jnp.float32)
    total = jnp.sum(accv) + jnp.max(tval)
    obuf[...] = jnp.full((_L,), total, jnp.float32)
    pltpu.sync_copy(obuf, out_hbm.at[wid])


def _phase1_call(cls_t, loc_t, landm_t, pri_t, tgt, nb):
    P = _P
    return pl.pallas_call(
        _phase1_body,
        grid=(nb,),
        in_specs=[
            pl.BlockSpec((1, _C, P), lambda b: (b, 0, 0)),
            pl.BlockSpec((1, 4, P), lambda b: (b, 0, 0)),
            pl.BlockSpec((1, 10, P), lambda b: (b, 0, 0)),
            pl.BlockSpec((4, P), lambda b: (0, 0)),
            pl.BlockSpec((1, _G, 15), lambda b: (b, 0, 0)),
        ],
        out_specs=[
            pl.BlockSpec((1, 1, P), lambda b: (b, 0, 0)),
            pl.BlockSpec((1, 1, 128), lambda b: (b, 0, 0)),
        ],
        out_shape=[
            jax.ShapeDtypeStruct((nb, 1, P), jnp.float32),
            jax.ShapeDtypeStruct((nb, 1, 128), jnp.float32),
        ],
        scratch_shapes=[pltpu.VMEM((32, P), jnp.float32)],
    )(cls_t, loc_t, landm_t, pri_t, tgt)


def _mine_call(vrows, smrows, nrows):
    mine = pl.kernel(
        functools.partial(_mine_body, nrows=nrows),
        mesh=plsc.VectorSubcoreMesh(core_axis_name="c", subcore_axis_name="s"),
        out_type=jax.ShapeDtypeStruct((nrows, _L), jnp.float32),
        compiler_params=pltpu.CompilerParams(needs_layout_passes=False),
        scratch_types=[
            pltpu.VMEM((_P,), jnp.float32),
            pltpu.VMEM((_P,), jnp.int32),
            pltpu.VMEM((128,), jnp.float32),
            pltpu.VMEM((2048,), jnp.int32),
            pltpu.VMEM((_L,), jnp.float32),
        ],
    )
    return mine(vrows, smrows)


@jax.jit
def kernel(cls_data, loc_data, landm_data, priors, targets):
    B, P = _B, _P
    H = B // 2
    cls_t = jnp.transpose(cls_data, (0, 2, 1))
    loc_t = jnp.transpose(loc_data, (0, 2, 1))
    landm_t = jnp.transpose(landm_data, (0, 2, 1))
    pri_t = jnp.transpose(priors, (1, 0))

    v, sums = _phase1_call(cls_t, loc_t, landm_t, pri_t, targets, B)
    rows = _mine_call(v.reshape(B, P), sums.reshape(B, 128), B)

    sm = sums.reshape(B, 128)
    n = jnp.maximum(jnp.sum(sm[:, 2]), 1.0)
    return (jnp.sum(sm[:, 0]) / n, jnp.sum(rows[:, 0]) / n,
            jnp.sum(sm[:, 1]) / n)


# speedup vs baseline: 1.1333x; 1.0089x over previous
import functools

import jax
import jax.numpy as jnp
from jax import lax
from jax.experimental import pallas as pl
from jax.experimental.pallas import tpu as pltpu
from jax.experimental.pallas import tpu_sc as plsc

_B, _P, _G, _C = 32, 16800, 16, 2
_TH = 0.35
_NEGPOS = 7
_V0, _V1 = 0.1, 0.2


def _smooth_l1(d):
    ad = jnp.abs(d)
    return jnp.where(ad < 1.0, 0.5 * d * d, ad - 0.5)


def _phase1_body(cls_ref, loc_ref, landm_ref, pri_ref, tgt_ref,
                 v_ref, sums_ref, s_ref):
    G, P = _G, _P

    @pl.when(pl.program_id(0) == 0)
    def _init():
        pri = pri_ref[...]
        pcx, pcy = pri[0:1], pri[1:2]
        pw, ph = pri[2:3], pri[3:4]
        s_ref[0:1] = pcx - pw * 0.5
        s_ref[1:2] = pcy - ph * 0.5
        s_ref[2:3] = pcx + pw * 0.5
        s_ref[3:4] = pcy + ph * 0.5
        s_ref[4:5] = pw * ph
        inv_vw = 1.0 / (_V0 * pw)
        inv_vh = 1.0 / (_V0 * ph)
        s_ref[5:6] = inv_vw
        s_ref[6:7] = inv_vh
        s_ref[7:8] = 1.0 / pw
        s_ref[8:9] = 1.0 / ph
        s_ref[9:19] = jnp.concatenate([pcx, pcy] * 5, axis=0)
        s_ref[19:29] = jnp.concatenate([inv_vw, inv_vh] * 5, axis=0)

    pri = pri_ref[...]
    pcx, pcy = pri[0:1], pri[1:2]
    pw, ph = pri[2:3], pri[3:4]
    px1, py1 = s_ref[0:1], s_ref[1:2]
    px2, py2 = s_ref[2:3], s_ref[3:4]

    tgt = tgt_ref[0]
    tx1, ty1 = tgt[:, 0:1], tgt[:, 1:2]
    tx2, ty2 = tgt[:, 2:3], tgt[:, 3:4]

    iw = jnp.maximum(jnp.minimum(tx2, px2) - jnp.maximum(tx1, px1), 0.0)
    ih = jnp.maximum(jnp.minimum(ty2, py2) - jnp.maximum(ty1, py1), 0.0)
    inter = iw * ih
    area_t = (tx2 - tx1) * (ty2 - ty1)
    ov = inter / (area_t + s_ref[4:5] - inter)

    gi = lax.broadcasted_iota(jnp.int32, (G, P), 0)
    pi = lax.broadcasted_iota(jnp.int32, (G, P), 1)

    bto = jnp.max(ov, axis=0, keepdims=True)
    bti = jnp.min(jnp.where(ov == bto, gi, G), axis=0, keepdims=True)
    bpo = jnp.max(ov, axis=1, keepdims=True)
    bpi = jnp.min(jnp.where(ov == bpo, pi, P), axis=1, keepdims=True)
    valid = bpo >= 0.2

    eqbp = bpi == pi
    genc = jnp.max(jnp.where(eqbp, 2 * gi + valid.astype(jnp.int32), -1),
                   axis=0, keepdims=True)
    has = genc >= 0
    gidx = lax.shift_right_logical(jnp.maximum(genc, 0), 1)
    bto2 = jnp.where(has & (jnp.bitwise_and(genc, 1) == 1), 2.0, bto)
    bti2 = jnp.where(has, gidx, bti)

    onehot = (gi == bti2).astype(jnp.float32)
    m = lax.dot_general(tgt, onehot, (((0,), (0,)), ((), ())),
                        precision=lax.Precision.HIGHEST,
                        preferred_element_type=jnp.float32)

    label = m[14:15]
    conf = jnp.where(bto2 < _TH, 0, label.astype(jnp.int32))
    posf = (conf != 0).astype(jnp.float32)

    mx1, my1, mx2, my2 = m[0:1], m[1:2], m[2:3], m[3:4]
    gcx = ((mx1 + mx2) * 0.5 - pcx) * s_ref[5:6]
    gcy = ((my1 + my2) * 0.5 - pcy) * s_ref[6:7]
    gw = jnp.log((mx2 - mx1) * s_ref[7:8]) * (1.0 / _V1)
    gh = jnp.log((my2 - my1) * s_ref[8:9]) * (1.0 / _V1)
    gloc = jnp.concatenate([gcx, gcy, gw, gh], axis=0)
    loss_l = jnp.sum(_smooth_l1(loc_ref[0] - gloc) * posf)

    glm = (m[4:14] - s_ref[9:19]) * s_ref[19:29]
    loss_lm = jnp.sum(_smooth_l1(landm_ref[0] - glm) * posf)

    cls0, cls1 = cls_ref[0, 0:1], cls_ref[0, 1:2]
    mc = jnp.maximum(cls0, cls1)
    lse = mc + jnp.log(jnp.exp(cls0 - mc) + jnp.exp(cls1 - mc))
    csel = jnp.where(conf == 0, cls0, cls1)
    v = lse - csel

    num_pos = jnp.sum(posf)

    v_ref[0] = jnp.where(conf != 0, -v, v)
    li = lax.broadcasted_iota(jnp.int32, (1, 128), 1)
    sums_ref[0] = jnp.where(
        li == 0, loss_l, jnp.where(li == 1, loss_lm,
                                   jnp.where(li == 2, num_pos, 0.0)))


_L = 16
_CH = _P // _L
_NC = 2


def _mine_body(v_hbm, sums_hbm, out_hbm, vbuf, vbbuf, sbuf, hist, obuf,
               nrows=_B):
    wid = lax.axis_index("s") * _NC + lax.axis_index("c")
    wid = lax.rem(wid, nrows)
    pltpu.sync_copy(v_hbm.at[wid], vbuf)
    pltpu.sync_copy(sums_hbm.at[wid], sbuf)

    k = jnp.minimum(sbuf[pl.ds(0, _L)][2].astype(jnp.int32) * _NEGPOS, _P - 1)

    zeros16 = jnp.zeros((_L,), jnp.int32)
    ones16 = jnp.ones((_L,), jnp.int32)
    lane = lax.iota(jnp.int32, _L)

    def zero_hist(nbins):
        def zb(j, c):
            hist[pl.ds(j * _L, _L)] = zeros16
            return c
        lax.fori_loop(0, nbins // _L, zb, 0)

    def scan_level(nbins, kr):
        nch = nbins // _L

        def st1(jj, c):
            acc, found, j_star, acc_b = c
            j = nch - 1 - jj
            ssum = jnp.sum(hist[pl.ds(j * _L, _L)])
            newacc = acc + ssum
            hit = jnp.logical_and(found == 0, newacc >= kr)
            j_star = jnp.where(hit, j, j_star)
            acc_b = jnp.where(hit, acc, acc_b)
            found = jnp.where(newacc >= kr, 1, found)
            return (newacc, found, j_star, acc_b)

        _, _, j_star, acc_b = lax.fori_loop(0, nch, st1, (0, 0, 0, 0))
        chr_ = lax.rev(hist[pl.ds(j_star * _L, _L)], (0,))
        cum = plsc.cumsum(chr_)
        mask = (acc_b + cum) >= kr
        l_star = jnp.max(plsc.all_reduce_ffs(mask))
        f_star = j_star * _L + (_L - 1) - l_star
        hsel = jnp.sum(jnp.where(lane == l_star, chr_, 0))
        csel = jnp.sum(jnp.where(lane == l_star, cum, 0))
        return f_star, acc_b + csel - hsel

    zero_hist(2048)

    def pa(i, c):
        for u in range(2):
            x = vbuf[pl.ds((2 * i + u) * _L, _L)]
            vb = jnp.bitwise_and(lax.bitcast_convert_type(x, jnp.int32),
                                 0x7FFFFFFF)
            vbbuf[pl.ds((2 * i + u) * _L, _L)] = vb
            plsc.addupdate_scatter(
                hist, [lax.shift_right_logical(vb, 20)], ones16)
        return c
    lax.fori_loop(0, _CH // 2, pa, 0)
    f1, above1 = scan_level(2048, k)

    zero_hist(2048)

    def pb(i, c):
        for u in range(2):
            vb = vbbuf[pl.ds((2 * i + u) * _L, _L)]
            m = lax.shift_right_logical(vb, 20) == f1
            b2 = jnp.bitwise_and(lax.shift_right_logical(vb, 9), 0x7FF)
            plsc.addupdate_scatter(hist, [b2], ones16, mask=m)
        return c
    lax.fori_loop(0, _CH // 2, pb, 0)
    f2, above2 = scan_level(2048, k - above1)
    prefix = jnp.bitwise_or(lax.shift_left(f1, 11), f2)

    zero_hist(512)

    def pc(i, c):
        for u in range(2):
            vb = vbbuf[pl.ds((2 * i + u) * _L, _L)]
            m = lax.shift_right_logical(vb, 9) == prefix
            b3 = jnp.bitwise_and(vb, 0x1FF)
            plsc.addupdate_scatter(hist, [b3], ones16, mask=m)
        return c
    lax.fori_loop(0, _CH // 2, pc, 0)
    f3, cnt_gt = scan_level(512, k - above1 - above2)
    cnt_gt = cnt_gt + above1 + above2
    t = jnp.bitwise_or(lax.shift_left(prefix, 9), f3)
    r = k - cnt_gt

    def pd(i, c):
        accv, tcnt, tnp = c
        for u in range(2):
            sl = pl.ds((2 * i + u) * _L, _L)
            raw = lax.bitcast_convert_type(vbuf[sl], jnp.int32)
            pos = raw < 0
            vb = vbbuf[sl]
            gt = vb > t
            tie = vb == t
            x = lax.bitcast_convert_type(vb, jnp.float32)
            accv = accv + jnp.where(jnp.logical_or(gt, pos), x, 0.0)
            tcnt = tcnt + jnp.where(tie, 1, 0)
            tnp = tnp + jnp.where(jnp.logical_and(tie, jnp.logical_not(pos)),
                                  1, 0)
        return accv, tcnt, tnp

    accv, tcnt, tnp = lax.fori_loop(
        0, _CH // 2, pd, (jnp.zeros((_L,), jnp.float32), zeros16, zeros16))
    m_ties = jnp.sum(tcnt)
    all_tnp = jnp.sum(tnp)

    def pe_loop():
        def pe(i, c):
            base, cnt = c
            vb = vbbuf[pl.ds(i * _L, _L)]
            pos = lax.bitcast_convert_type(vbuf[pl.ds(i * _L, _L)],
                                           jnp.int32) < 0
            ti = jnp.where(vb == t, 1, 0)
            excl = plsc.cumsum(ti) - ti
            seltie = jnp.logical_and(
                jnp.logical_and(ti == 1, base + excl < r),
                jnp.logical_not(pos))
            cnt = cnt + jnp.sum(jnp.where(seltie, 1, 0))
            base = base + jnp.sum(ti)
            return base, cnt
        _, cnt = lax.fori_loop(0, _CH, pe, (0, 0))
        return cnt

    n_tie_nonpos = lax.cond(r == m_ties, lambda: all_tnp, pe_loop)

    tf = jnp.full((_L,), t, jnp.int32)
    tval = lax.bitcast_convert_type(tf, jnp.float32) * n_tie_nonpos.astype(jnp.float32)
    total = jnp.sum(accv) + jnp.max(tval)
    obuf[...] = jnp.full((_L,), total, jnp.float32)
    pltpu.sync_copy(obuf, out_hbm.at[wid])


def _phase1_call(cls_t, loc_t, landm_t, pri_t, tgt, nb):
    P = _P
    return pl.pallas_call(
        _phase1_body,
        grid=(nb,),
        in_specs=[
            pl.BlockSpec((1, _C, P), lambda b: (b, 0, 0)),
            pl.BlockSpec((1, 4, P), lambda b: (b, 0, 0)),
            pl.BlockSpec((1, 10, P), lambda b: (b, 0, 0)),
            pl.BlockSpec((4, P), lambda b: (0, 0)),
            pl.BlockSpec((1, _G, 15), lambda b: (b, 0, 0)),
        ],
        out_specs=[
            pl.BlockSpec((1, 1, P), lambda b: (b, 0, 0)),
            pl.BlockSpec((1, 1, 128), lambda b: (b, 0, 0)),
        ],
        out_shape=[
            jax.ShapeDtypeStruct((nb, 1, P), jnp.float32),
            jax.ShapeDtypeStruct((nb, 1, 128), jnp.float32),
        ],
        scratch_shapes=[pltpu.VMEM((32, P), jnp.float32)],
    )(cls_t, loc_t, landm_t, pri_t, tgt)


def _mine_call(vrows, smrows, nrows):
    mine = pl.kernel(
        functools.partial(_mine_body, nrows=nrows),
        mesh=plsc.VectorSubcoreMesh(core_axis_name="c", subcore_axis_name="s"),
        out_type=jax.ShapeDtypeStruct((nrows, _L), jnp.float32),
        compiler_params=pltpu.CompilerParams(needs_layout_passes=False),
        scratch_types=[
            pltpu.VMEM((_P,), jnp.float32),
            pltpu.VMEM((_P,), jnp.int32),
            pltpu.VMEM((128,), jnp.float32),
            pltpu.VMEM((2048,), jnp.int32),
            pltpu.VMEM((_L,), jnp.float32),
        ],
    )
    return mine(vrows, smrows)


@jax.jit
def kernel(cls_data, loc_data, landm_data, priors, targets):
    B, P = _B, _P
    H = B // 2
    cls_t = jnp.transpose(cls_data, (0, 2, 1))
    loc_t = jnp.transpose(loc_data, (0, 2, 1))
    landm_t = jnp.transpose(landm_data, (0, 2, 1))
    pri_t = jnp.transpose(priors, (1, 0))

    v, sums = _phase1_call(cls_t, loc_t, landm_t, pri_t, targets, B)
    rows = _mine_call(v.reshape(B, P), sums.reshape(B, 128), B)

    sm = sums.reshape(B, 128)
    n = jnp.maximum(jnp.sum(sm[:, 2]), 1.0)
    return (jnp.sum(sm[:, 0]) / n, jnp.sum(rows[:, 0]) / n,
            jnp.sum(sm[:, 1]) / n)
